# baseline (speedup 1.0000x reference)
import jax
import jax.numpy as jnp
from jax.experimental import pallas as pl
from jax.experimental.pallas import tpu as pltpu

_C1 = 32
_NB = 16


def _conv_kernel(x_ref, w1_ref, b1_ref, w2_ref, b2_ref, o_ref,
                 pad1_ref, pad2_ref):
    nb, H, W, Cin = x_ref.shape
    HW = H * W
    pad1_ref[...] = jnp.zeros_like(pad1_ref)
    pad1_ref[:, 1:H + 1, 1:W + 1, :] = x_ref[...]
    p1 = jnp.concatenate(
        [pad1_ref[:, dy:dy + H, dx:dx + W, :].reshape(nb * HW, Cin)
         for dy in range(3) for dx in range(3)], axis=-1)
    h1 = jnp.dot(p1, w1_ref[...], preferred_element_type=jnp.float32)
    h1 = jnp.maximum(h1 + b1_ref[...], 0.0).astype(jnp.bfloat16)
    pad2_ref[...] = jnp.zeros_like(pad2_ref)
    pad2_ref[:, 1:H + 1, 1:W + 1, :] = h1.reshape(nb, H, W, _C1)
    p2 = jnp.concatenate(
        [pad2_ref[:, dy:dy + H, dx:dx + W, :].reshape(nb * HW, _C1)
         for dy in range(3) for dx in range(3)], axis=-1)
    h2 = jnp.dot(p2, w2_ref[...], preferred_element_type=jnp.float32)
    h2 = jnp.maximum(h2 + b2_ref[...], 0.0).astype(jnp.bfloat16)
    o_ref[...] = h2.reshape(nb, HW, _C1)


def _conv_stack(x_nhwc, w1, b1, w2, b2):
    B, H, W, Cin = x_nhwc.shape
    nb = _NB if B % _NB == 0 else 1
    return pl.pallas_call(
        _conv_kernel,
        out_shape=jax.ShapeDtypeStruct((B, H * W, _C1), jnp.bfloat16),
        grid=(B // nb,),
        in_specs=[
            pl.BlockSpec((nb, H, W, Cin), lambda b: (b, 0, 0, 0)),
            pl.BlockSpec((9 * Cin, _C1), lambda b: (0, 0)),
            pl.BlockSpec((1, _C1), lambda b: (0, 0)),
            pl.BlockSpec((9 * _C1, _C1), lambda b: (0, 0)),
            pl.BlockSpec((1, _C1), lambda b: (0, 0)),
        ],
        out_specs=pl.BlockSpec((nb, H * W, _C1), lambda b: (b, 0, 0)),
        scratch_shapes=[
            pltpu.VMEM((nb, H + 2, W + 2, Cin), jnp.bfloat16),
            pltpu.VMEM((nb, H + 2, W + 2, _C1), jnp.bfloat16),
        ],
        compiler_params=pltpu.CompilerParams(
            dimension_semantics=("parallel",)),
    )(x_nhwc, w1, b1, w2, b2)


def _fc_kernel(x_ref, w_ref, b_ref, o_ref):
    k = pl.program_id(1)
    acc = jnp.dot(x_ref[...], w_ref[...], preferred_element_type=jnp.float32)

    @pl.when(k == 0)
    def _init():
        o_ref[...] = acc + b_ref[...]

    @pl.when(k != 0)
    def _accum():
        o_ref[...] += acc


def _fc(x, w_kn, b_1n):
    B, K = x.shape
    N = w_kn.shape[1]
    bm = B // 2 if B % 2 == 0 else B
    bk = 4096 if K % 4096 == 0 else K
    return pl.pallas_call(
        _fc_kernel,
        out_shape=jax.ShapeDtypeStruct((B, N), jnp.float32),
        grid=(B // bm, K // bk),
        in_specs=[
            pl.BlockSpec((bm, bk), lambda m, k: (m, k)),
            pl.BlockSpec((bk, N), lambda m, k: (k, 0)),
            pl.BlockSpec((1, N), lambda m, k: (0, 0)),
        ],
        out_specs=pl.BlockSpec((bm, N), lambda m, k: (m, 0)),
        compiler_params=pltpu.CompilerParams(
            dimension_semantics=("parallel", "arbitrary")),
    )(x, w_kn, b_1n)


def kernel(w1, b1, w2, b2, fc_w, fc_b, x_nchw):
    if x_nchw.ndim == 3:
        x_nchw = x_nchw[None]
    x = jnp.transpose(x_nchw, (0, 2, 3, 1)).astype(jnp.bfloat16)
    h = _conv_stack(x, w1.astype(jnp.bfloat16), b1,
                    w2.astype(jnp.bfloat16), b2)
    h = h.reshape(h.shape[0], -1)
    return _fc(h, fc_w.astype(jnp.bfloat16), fc_b)
